# Initial kernel scaffold; baseline (speedup 1.0000x reference)
import jax
import jax.numpy as jnp
from jax.experimental import pallas as pl


def kernel(idx, targets, table):
    raise NotImplementedError("write your pallas kernel here")


# speedup vs baseline: 1.2259x; 1.2259x over previous
import functools

import jax
import jax.numpy as jnp
from jax import lax
from jax.experimental import pallas as pl
from jax.experimental.pallas import tpu as pltpu
from jax.experimental.pallas import tpu_sc as plsc

_NC = 2
_NS = 16
_NW = _NC * _NS
_K = 80


def _lse_body(table_ref, lse_ref):
    x = table_ref[...]
    m = jnp.max(x, axis=1)
    s = jnp.sum(jnp.exp(x - m[:, None]), axis=1)
    lse_ref[...] = m + jnp.log(s)


def _row_lse(table):
    v = table.shape[0]
    return pl.pallas_call(
        _lse_body,
        out_shape=jax.ShapeDtypeStruct((v,), jnp.float32),
    )(table)


def _make_sc_gather(n, v, c):
    rows_per_w = n // _NW
    n_chunks = rows_per_w // _K
    mesh = plsc.VectorSubcoreMesh(core_axis_name="c", subcore_axis_name="s")

    @functools.partial(
        pl.kernel,
        out_type=[
            jax.ShapeDtypeStruct((n, c), jnp.float32),
            jax.ShapeDtypeStruct((_NW, 16), jnp.float32),
        ],
        mesh=mesh,
        compiler_params=pltpu.CompilerParams(use_tc_tiling_on_sc=False,
                                             needs_layout_passes=False),
        scratch_types=[
            pltpu.VMEM((rows_per_w,), jnp.int32),
            pltpu.VMEM((rows_per_w,), jnp.int32),
            pltpu.VMEM((v,), jnp.float32),
            pltpu.VMEM((_K, c), jnp.float32),
            pltpu.VMEM((16,), jnp.float32),
            pltpu.SemaphoreType.DMA,
        ],
    )
    def sc_kernel(table_hbm, idx_hbm, tgt_hbm, lse_hbm, out_hbm, part_hbm,
                  idx_v, tgt_v, lse_v, rows_v, acc_v, sem):
        wid = lax.axis_index("s") * _NC + lax.axis_index("c")
        base = wid * rows_per_w
        pltpu.sync_copy(idx_hbm.at[pl.ds(base, rows_per_w)], idx_v)
        pltpu.sync_copy(tgt_hbm.at[pl.ds(base, rows_per_w)], tgt_v)
        pltpu.sync_copy(lse_hbm, lse_v)
        iota16 = lax.iota(jnp.int32, 16)

        def chunk_body(ci, acc):
            off = ci * _K
            pltpu.async_copy(table_hbm.at[idx_v.at[pl.ds(off, _K)]],
                             rows_v, sem).wait()
            pltpu.sync_copy(rows_v, out_hbm.at[pl.ds(base + off, _K)])
            for j in range(_K // 16):
                rloc = iota16 + j * 16
                tg = tgt_v[pl.ds(off + j * 16, 16)]
                iv = idx_v[pl.ds(off + j * 16, 16)]
                tv = plsc.load_gather(rows_v, [rloc, tg])
                lv = plsc.load_gather(lse_v, [iv])
                acc = acc + (lv - tv)
            return acc

        acc = lax.fori_loop(0, n_chunks, chunk_body,
                            jnp.zeros((16,), jnp.float32))
        acc_v[...] = acc
        pltpu.sync_copy(acc_v, part_hbm.at[wid])

    return sc_kernel


def kernel(idx, targets, table):
    b, t = idx.shape
    v, c = table.shape
    n = b * t
    idx_f = idx.reshape(n).astype(jnp.int32)
    tgt_f = targets.reshape(n).astype(jnp.int32)
    row_lse = _row_lse(table)
    logits_flat, partials = _make_sc_gather(n, v, c)(
        table, idx_f, tgt_f, row_lse)
    loss = jnp.sum(partials) / n
    return (logits_flat, loss)
